# baseline (speedup 1.0000x reference)
import functools

import jax
import jax.numpy as jnp
from jax import lax
from jax.experimental import pallas as pl
from jax.experimental.pallas import tpu as pltpu
from jax.experimental.pallas import tpu_sc as plsc

_N = 10000
_N2 = 10240
_D = 128
_E = 320000
_NC = 2
_NS = 16
_NW = _NC * _NS
_EPW = _E // _NW
_CH = 80
_SB = 2000
_NSB = _EPW // _SB
_CPS = _SB // _CH
_RPT = _N2 // _NS
_L = 16


def _lane_bcast(v, l):
  idx = jnp.full((_L, 1), l, jnp.int32)
  dn = lax.GatherDimensionNumbers(
      offset_dims=(), collapsed_slice_dims=(0,), start_index_map=(0,))
  return lax.gather(v, idx, dn, (1,),
                    mode=lax.GatherScatterMode.PROMISE_IN_BOUNDS)


def _sc_body(src_hbm, dst_hbm, vals_hbm, ego_hbm, out0_hbm, out1_hbm,
             src_v, dst_v, vals_v, ga, gb, sa, sb_, acc, gs0, gs1, ss0, ss1):
  cid = lax.axis_index("c")
  sid = lax.axis_index("s")
  wid = cid * _NS + sid

  def start_gather(c, buf, sem):
    return pltpu.async_copy(
        ego_hbm.at[src_v.at[pl.ds(c * _CH, _CH)]], buf, sem)

  def wait_gather(c, buf, sem):
    pltpu.make_async_copy(
        ego_hbm.at[src_v.at[pl.ds(c * _CH, _CH)]], buf, sem).wait()

  def start_scatter(c, buf, sem):
    return pltpu.async_copy(buf, acc.at[dst_v.at[c]], sem, add=True)

  def wait_scatter(c, buf, sem):
    pltpu.make_async_copy(buf, acc.at[dst_v.at[c]], sem).wait()

  def scale(gbuf, sbuf, c):
    def group(g, carry):
      vv = vals_v[pl.ds(c * _CH + g * _L, _L)]
      e0 = g * _L
      bs = [_lane_bcast(vv, l) for l in range(_L)]

      def kslice(k, carry2):
        for l in range(_L):
          sbuf[e0 + l, pl.ds(k * _L, _L)] = (
              gbuf[e0 + l, pl.ds(k * _L, _L)] * bs[l])
        return carry2

      lax.fori_loop(0, _D // _L, kslice, 0)
      return carry

    lax.fori_loop(0, _CH // _L, group, 0)

  z = jnp.zeros((_L,), jnp.float32)

  def zrow(i, carry):
    for k in range(_D // _L):
      ga[i, pl.ds(k * _L, _L)] = z
    return carry

  lax.fori_loop(0, _CH, zrow, 0)
  for k in range(_RPT // _CH):
    r0 = sid * _RPT + k * _CH

    @pl.when(r0 < _N)
    def _():
      pltpu.sync_copy(ga, acc.at[pl.ds(r0, _CH)])
  plsc.subcore_barrier()

  def superblock(sb, carry0):
    base = wid * _EPW + sb * _SB
    pltpu.sync_copy(src_hbm.at[pl.ds(base, _SB)], src_v)
    pltpu.sync_copy(vals_hbm.at[pl.ds(base, _SB)], vals_v)
    pltpu.sync_copy(dst_hbm.at[wid * _NSB + sb], dst_v)

    start_gather(0, ga, gs0)
    start_gather(1, gb, gs1)

    def pair(pp, carry):
      c0 = 2 * pp
      c1 = c0 + 1
      wait_gather(c0, ga, gs0)

      @pl.when(pp > 0)
      def _():
        wait_scatter(c0 - 2, sa, ss0)

      scale(ga, sa, c0)
      start_scatter(c0, sa, ss0)
      start_gather(c0 + 2, ga, gs0)

      wait_gather(c1, gb, gs1)

      @pl.when(pp > 0)
      def _():
        wait_scatter(c1 - 2, sb_, ss1)

      scale(gb, sb_, c1)
      start_scatter(c1, sb_, ss1)

      @pl.when(pp < _CPS // 2 - 1)
      def _():
        start_gather(c1 + 2, gb, gs1)

      return carry

    lax.fori_loop(0, _CPS // 2, pair, 0)
    wait_gather(_CPS - 1, ga, gs0)
    wait_scatter(_CPS - 3, sa, ss0)
    scale(ga, sa, _CPS - 1)
    start_scatter(_CPS - 1, sa, ss0)
    wait_scatter(_CPS - 2, sb_, ss1)
    wait_scatter(_CPS - 1, sa, ss0)
    return carry0

  lax.fori_loop(0, _NSB, superblock, 0)
  plsc.subcore_barrier()

  def dump(oref):
    for k in range(_RPT // _CH):
      r0 = sid * _RPT + k * _CH

      @pl.when(r0 < _N)
      def _():
        pltpu.sync_copy(acc.at[pl.ds(r0, _CH)], ga)
        pltpu.sync_copy(ga, oref.at[pl.ds(r0, _CH)])

  @pl.when(cid == 0)
  def _():
    dump(out0_hbm)

  @pl.when(cid == 1)
  def _():
    dump(out1_hbm)


def _sc_segment_sum(src, dst3d, vals, ego):
  mesh = plsc.VectorSubcoreMesh(core_axis_name="c", subcore_axis_name="s")
  return pl.kernel(
      _sc_body,
      out_type=(jax.ShapeDtypeStruct((_N, _D), jnp.float32),
                jax.ShapeDtypeStruct((_N, _D), jnp.float32)),
      mesh=mesh,
      scratch_types=[
          pltpu.VMEM((_SB,), jnp.int32),
          pltpu.VMEM((_CPS, _CH), jnp.int32),
          pltpu.VMEM((_SB,), jnp.float32),
          pltpu.VMEM((_CH, _D), jnp.float32),
          pltpu.VMEM((_CH, _D), jnp.float32),
          pltpu.VMEM((_CH, _D), jnp.float32),
          pltpu.VMEM((_CH, _D), jnp.float32),
          pltpu.VMEM_SHARED((_N, _D), jnp.float32),
          pltpu.SemaphoreType.DMA,
          pltpu.SemaphoreType.DMA,
          pltpu.SemaphoreType.DMA,
          pltpu.SemaphoreType.DMA,
      ],
  )(src, dst3d, vals, ego)


_BN = 1000


def _tc_body(p0_ref, p1_ref, ego_ref, wgc_ref, bgc_ref, wbi_ref, bbi_ref,
             out_ref):
  side = p0_ref[...] + p1_ref[...]
  gcn = jnp.dot(side, wgc_ref[...],
                preferred_element_type=jnp.float32) + bgc_ref[...]
  gcn = jnp.where(gcn >= 0, gcn, 0.2 * gcn)
  bi = jnp.dot(ego_ref[...] * side, wbi_ref[...],
               preferred_element_type=jnp.float32) + bbi_ref[...]
  bi = jnp.where(bi >= 0, bi, 0.2 * bi)
  o = gcn + bi
  ss = jnp.sum(o * o, axis=1, keepdims=True)
  out_ref[...] = o / jnp.sqrt(jnp.maximum(ss, 1e-12))


def _tc_mlp(p0, p1, ego, w_gc, b_gc, w_bi, b_bi):
  row_spec = pl.BlockSpec((_BN, _D), lambda i: (i, 0))
  full_w = pl.BlockSpec((_D, _D), lambda i: (0, 0))
  full_b = pl.BlockSpec((1, _D), lambda i: (0, 0))
  return pl.pallas_call(
      _tc_body,
      grid=(_N // _BN,),
      in_specs=[row_spec, row_spec, row_spec, full_w, full_b, full_w, full_b],
      out_specs=row_spec,
      out_shape=jax.ShapeDtypeStruct((_N, _D), jnp.float32),
  )(p0, p1, ego, w_gc, b_gc, w_bi, b_bi)


@jax.jit
def kernel(edge_index, edge_vals, ego_embeddings, w_gc, b_gc, w_bi, b_bi):
  src = edge_index[0].astype(jnp.int32)
  dst3d = edge_index[1].astype(jnp.int32).reshape(_NW * _NSB, _CPS, _CH)
  p0, p1 = _sc_segment_sum(src, dst3d, edge_vals, ego_embeddings)
  return _tc_mlp(p0, p1, ego_embeddings,
                 w_gc, b_gc.reshape(1, _D), w_bi, b_bi.reshape(1, _D))


# speedup vs baseline: 2.6321x; 2.6321x over previous
import functools

import jax
import jax.numpy as jnp
from jax import lax
from jax.experimental import pallas as pl
from jax.experimental.pallas import tpu as pltpu
from jax.experimental.pallas import tpu_sc as plsc

_N = 10000
_N2 = 10240
_D = 128
_E = 320000
_NC = 2
_NS = 16
_NW = _NC * _NS
_EPW = _E // _NW
_CH = 80
_SB = 2000
_NSB = _EPW // _SB
_CPS = _SB // _CH
_RPT = _N2 // _NS
_L = 16


def _lane_bcast(v, l):
  idx = jnp.full((_L, 1), l, jnp.int32)
  dn = lax.GatherDimensionNumbers(
      offset_dims=(), collapsed_slice_dims=(0,), start_index_map=(0,))
  return lax.gather(v, idx, dn, (1,),
                    mode=lax.GatherScatterMode.PROMISE_IN_BOUNDS)


def _sc_body(src_hbm, dst_hbm, vals_hbm, ego_hbm, out0_hbm, out1_hbm,
             src_v, dst_v, vals_v, ga, gb, sa, sb_, acc, gs0, gs1, ss0, ss1):
  cid = lax.axis_index("c")
  sid = lax.axis_index("s")
  wid = cid * _NS + sid

  def start_gather(c, buf, sem):
    return pltpu.async_copy(
        ego_hbm.at[src_v.at[pl.ds(c * _CH, _CH)]], buf, sem)

  def wait_gather(c, buf, sem):
    pltpu.make_async_copy(
        ego_hbm.at[src_v.at[pl.ds(c * _CH, _CH)]], buf, sem).wait()

  def start_scatter(c, buf, sem):
    return pltpu.async_copy(buf, acc.at[dst_v.at[c]], sem, add=True)

  def wait_scatter(c, buf, sem):
    pltpu.make_async_copy(buf, acc.at[dst_v.at[c]], sem).wait()

  def scale(gbuf, sbuf, c):
    def make_group(l0):
      def group(g, carry):
        vv = vals_v[pl.ds(c * _CH + g * _L, _L)]
        e0 = g * _L
        for l in range(l0, l0 + _L // 2):
          b = _lane_bcast(vv, l)
          for k in range(_D // _L):
            sbuf[e0 + l, pl.ds(k * _L, _L)] = (
                gbuf[e0 + l, pl.ds(k * _L, _L)] * b)
        return carry
      return group

    lax.fori_loop(0, _CH // _L, make_group(0), 0)
    lax.fori_loop(0, _CH // _L, make_group(_L // 2), 0)

  z = jnp.zeros((_L,), jnp.float32)

  def zrow(i, carry):
    for k in range(_D // _L):
      ga[i, pl.ds(k * _L, _L)] = z
    return carry

  lax.fori_loop(0, _CH, zrow, 0)
  for k in range(_RPT // _CH):
    r0 = sid * _RPT + k * _CH

    @pl.when(r0 < _N)
    def _():
      pltpu.sync_copy(ga, acc.at[pl.ds(r0, _CH)])
  plsc.subcore_barrier()

  def superblock(sb, carry0):
    base = wid * _EPW + sb * _SB
    pltpu.sync_copy(src_hbm.at[pl.ds(base, _SB)], src_v)
    pltpu.sync_copy(vals_hbm.at[pl.ds(base, _SB)], vals_v)
    pltpu.sync_copy(dst_hbm.at[wid * _NSB + sb], dst_v)

    start_gather(0, ga, gs0)
    start_gather(1, gb, gs1)

    def pair(pp, carry):
      c0 = 2 * pp
      c1 = c0 + 1
      wait_gather(c0, ga, gs0)

      @pl.when(pp > 0)
      def _():
        wait_scatter(c0 - 2, sa, ss0)

      scale(ga, sa, c0)
      start_scatter(c0, sa, ss0)
      start_gather(c0 + 2, ga, gs0)

      wait_gather(c1, gb, gs1)

      @pl.when(pp > 0)
      def _():
        wait_scatter(c1 - 2, sb_, ss1)

      scale(gb, sb_, c1)
      start_scatter(c1, sb_, ss1)

      @pl.when(pp < _CPS // 2 - 1)
      def _():
        start_gather(c1 + 2, gb, gs1)

      return carry

    lax.fori_loop(0, _CPS // 2, pair, 0)
    wait_gather(_CPS - 1, ga, gs0)
    wait_scatter(_CPS - 3, sa, ss0)
    scale(ga, sa, _CPS - 1)
    start_scatter(_CPS - 1, sa, ss0)
    wait_scatter(_CPS - 2, sb_, ss1)
    wait_scatter(_CPS - 1, sa, ss0)
    return carry0

  lax.fori_loop(0, _NSB, superblock, 0)
  plsc.subcore_barrier()

  def dump(oref):
    for k in range(_RPT // _CH):
      r0 = sid * _RPT + k * _CH

      @pl.when(r0 < _N)
      def _():
        pltpu.sync_copy(acc.at[pl.ds(r0, _CH)], ga)
        pltpu.sync_copy(ga, oref.at[pl.ds(r0, _CH)])

  @pl.when(cid == 0)
  def _():
    dump(out0_hbm)

  @pl.when(cid == 1)
  def _():
    dump(out1_hbm)


def _sc_segment_sum(src, dst3d, vals, ego):
  mesh = plsc.VectorSubcoreMesh(core_axis_name="c", subcore_axis_name="s")
  return pl.kernel(
      _sc_body,
      out_type=(jax.ShapeDtypeStruct((_N, _D), jnp.float32),
                jax.ShapeDtypeStruct((_N, _D), jnp.float32)),
      mesh=mesh,
      scratch_types=[
          pltpu.VMEM((_SB,), jnp.int32),
          pltpu.VMEM((_CPS, _CH), jnp.int32),
          pltpu.VMEM((_SB,), jnp.float32),
          pltpu.VMEM((_CH, _D), jnp.float32),
          pltpu.VMEM((_CH, _D), jnp.float32),
          pltpu.VMEM((_CH, _D), jnp.float32),
          pltpu.VMEM((_CH, _D), jnp.float32),
          pltpu.VMEM_SHARED((_N, _D), jnp.float32),
          pltpu.SemaphoreType.DMA,
          pltpu.SemaphoreType.DMA,
          pltpu.SemaphoreType.DMA,
          pltpu.SemaphoreType.DMA,
      ],
  )(src, dst3d, vals, ego)


_BN = 1000


def _tc_body(p0_ref, p1_ref, ego_ref, wgc_ref, bgc_ref, wbi_ref, bbi_ref,
             out_ref):
  side = p0_ref[...] + p1_ref[...]
  gcn = jnp.dot(side, wgc_ref[...],
                preferred_element_type=jnp.float32) + bgc_ref[...]
  gcn = jnp.where(gcn >= 0, gcn, 0.2 * gcn)
  bi = jnp.dot(ego_ref[...] * side, wbi_ref[...],
               preferred_element_type=jnp.float32) + bbi_ref[...]
  bi = jnp.where(bi >= 0, bi, 0.2 * bi)
  o = gcn + bi
  ss = jnp.sum(o * o, axis=1, keepdims=True)
  out_ref[...] = o / jnp.sqrt(jnp.maximum(ss, 1e-12))


def _tc_mlp(p0, p1, ego, w_gc, b_gc, w_bi, b_bi):
  row_spec = pl.BlockSpec((_BN, _D), lambda i: (i, 0))
  full_w = pl.BlockSpec((_D, _D), lambda i: (0, 0))
  full_b = pl.BlockSpec((1, _D), lambda i: (0, 0))
  return pl.pallas_call(
      _tc_body,
      grid=(_N // _BN,),
      in_specs=[row_spec, row_spec, row_spec, full_w, full_b, full_w, full_b],
      out_specs=row_spec,
      out_shape=jax.ShapeDtypeStruct((_N, _D), jnp.float32),
  )(p0, p1, ego, w_gc, b_gc, w_bi, b_bi)


@jax.jit
def kernel(edge_index, edge_vals, ego_embeddings, w_gc, b_gc, w_bi, b_bi):
  src = edge_index[0].astype(jnp.int32)
  dst3d = edge_index[1].astype(jnp.int32).reshape(_NW * _NSB, _CPS, _CH)
  p0, p1 = _sc_segment_sum(src, dst3d, edge_vals, ego_embeddings)
  return _tc_mlp(p0, p1, ego_embeddings,
                 w_gc, b_gc.reshape(1, _D), w_bi, b_bi.reshape(1, _D))


# speedup vs baseline: 2.6348x; 1.0010x over previous
import functools

import jax
import jax.numpy as jnp
from jax import lax
from jax.experimental import pallas as pl
from jax.experimental.pallas import tpu as pltpu
from jax.experimental.pallas import tpu_sc as plsc

_N = 10000
_N2 = 10240
_D = 128
_E = 320000
_NC = 2
_NS = 16
_NW = _NC * _NS
_EPW = _E // _NW
_CH = 80
_SB = 2000
_NSB = _EPW // _SB
_CPS = _SB // _CH
_RPT = _N2 // _NS
_L = 16


def _lane_bcast(v, l):
  idx = jnp.full((_L, 1), l, jnp.int32)
  dn = lax.GatherDimensionNumbers(
      offset_dims=(), collapsed_slice_dims=(0,), start_index_map=(0,))
  return lax.gather(v, idx, dn, (1,),
                    mode=lax.GatherScatterMode.PROMISE_IN_BOUNDS)


def _sc_body(src_hbm, edge_hbm, vals_hbm, ego_hbm, out0_hbm, out1_hbm,
             src_v, dst_v, vals_v, ga, gb, sa, sb_, acc, gs0, gs1, ss0, ss1):
  cid = lax.axis_index("c")
  sid = lax.axis_index("s")
  wid = cid * _NS + sid

  def start_gather(c, buf, sem):
    return pltpu.async_copy(
        ego_hbm.at[src_v.at[pl.ds(c * _CH, _CH)]], buf, sem)

  def wait_gather(c, buf, sem):
    pltpu.make_async_copy(
        ego_hbm.at[src_v.at[pl.ds(c * _CH, _CH)]], buf, sem).wait()

  def start_scatter(c, buf, sem):
    return pltpu.async_copy(buf, acc.at[dst_v.at[c]], sem, add=True)

  def wait_scatter(c, buf, sem):
    pltpu.make_async_copy(buf, acc.at[dst_v.at[c]], sem).wait()

  def scale(gbuf, sbuf, c):
    def make_group(l0):
      def group(g, carry):
        vv = vals_v[pl.ds(c * _CH + g * _L, _L)]
        e0 = g * _L
        for l in range(l0, l0 + _L // 2):
          b = _lane_bcast(vv, l)
          for k in range(_D // _L):
            sbuf[e0 + l, pl.ds(k * _L, _L)] = (
                gbuf[e0 + l, pl.ds(k * _L, _L)] * b)
        return carry
      return group

    lax.fori_loop(0, _CH // _L, make_group(0), 0)
    lax.fori_loop(0, _CH // _L, make_group(_L // 2), 0)

  z = jnp.zeros((_L,), jnp.float32)

  def zrow(i, carry):
    for k in range(_D // _L):
      ga[i, pl.ds(k * _L, _L)] = z
    return carry

  lax.fori_loop(0, _CH, zrow, 0)
  for k in range(_RPT // _CH):
    r0 = sid * _RPT + k * _CH

    @pl.when(r0 < _N)
    def _():
      pltpu.sync_copy(ga, acc.at[pl.ds(r0, _CH)])
  plsc.subcore_barrier()

  def superblock(sb, carry0):
    base = wid * _EPW + sb * _SB
    pltpu.sync_copy(src_hbm.at[pl.ds(base, _SB)], src_v)
    pltpu.sync_copy(vals_hbm.at[pl.ds(base, _SB)], vals_v)
    pltpu.sync_copy(edge_hbm.at[1, wid * _NSB + sb], dst_v)

    start_gather(0, ga, gs0)
    start_gather(1, gb, gs1)

    def pair(pp, carry):
      c0 = 2 * pp
      c1 = c0 + 1
      wait_gather(c0, ga, gs0)

      @pl.when(pp > 0)
      def _():
        wait_scatter(c0 - 2, sa, ss0)

      scale(ga, sa, c0)
      start_scatter(c0, sa, ss0)
      start_gather(c0 + 2, ga, gs0)

      wait_gather(c1, gb, gs1)

      @pl.when(pp > 0)
      def _():
        wait_scatter(c1 - 2, sb_, ss1)

      scale(gb, sb_, c1)
      start_scatter(c1, sb_, ss1)

      @pl.when(pp < _CPS // 2 - 1)
      def _():
        start_gather(c1 + 2, gb, gs1)

      return carry

    lax.fori_loop(0, _CPS // 2, pair, 0)
    wait_gather(_CPS - 1, ga, gs0)
    wait_scatter(_CPS - 3, sa, ss0)
    scale(ga, sa, _CPS - 1)
    start_scatter(_CPS - 1, sa, ss0)
    wait_scatter(_CPS - 2, sb_, ss1)
    wait_scatter(_CPS - 1, sa, ss0)
    return carry0

  lax.fori_loop(0, _NSB, superblock, 0)
  plsc.subcore_barrier()

  def dump(oref):
    for k in range(_RPT // _CH):
      r0 = sid * _RPT + k * _CH

      @pl.when(r0 < _N)
      def _():
        pltpu.sync_copy(acc.at[pl.ds(r0, _CH)], ga)
        pltpu.sync_copy(ga, oref.at[pl.ds(r0, _CH)])

  @pl.when(cid == 0)
  def _():
    dump(out0_hbm)

  @pl.when(cid == 1)
  def _():
    dump(out1_hbm)


def _sc_segment_sum(src, edge4d, vals, ego):
  mesh = plsc.VectorSubcoreMesh(core_axis_name="c", subcore_axis_name="s")
  return pl.kernel(
      _sc_body,
      out_type=(jax.ShapeDtypeStruct((_N, _D), jnp.float32),
                jax.ShapeDtypeStruct((_N, _D), jnp.float32)),
      mesh=mesh,
      scratch_types=[
          pltpu.VMEM((_SB,), jnp.int32),
          pltpu.VMEM((_CPS, _CH), jnp.int32),
          pltpu.VMEM((_SB,), jnp.float32),
          pltpu.VMEM((_CH, _D), jnp.float32),
          pltpu.VMEM((_CH, _D), jnp.float32),
          pltpu.VMEM((_CH, _D), jnp.float32),
          pltpu.VMEM((_CH, _D), jnp.float32),
          pltpu.VMEM_SHARED((_N, _D), jnp.float32),
          pltpu.SemaphoreType.DMA,
          pltpu.SemaphoreType.DMA,
          pltpu.SemaphoreType.DMA,
          pltpu.SemaphoreType.DMA,
      ],
  )(src, edge4d, vals, ego)


_BN = 2000


def _tc_body(p0_ref, p1_ref, ego_ref, wgc_ref, bgc_ref, wbi_ref, bbi_ref,
             out_ref):
  side = p0_ref[...] + p1_ref[...]
  gcn = jnp.dot(side, wgc_ref[...],
                preferred_element_type=jnp.float32) + bgc_ref[...]
  gcn = jnp.where(gcn >= 0, gcn, 0.2 * gcn)
  bi = jnp.dot(ego_ref[...] * side, wbi_ref[...],
               preferred_element_type=jnp.float32) + bbi_ref[...]
  bi = jnp.where(bi >= 0, bi, 0.2 * bi)
  o = gcn + bi
  ss = jnp.sum(o * o, axis=1, keepdims=True)
  out_ref[...] = o / jnp.sqrt(jnp.maximum(ss, 1e-12))


def _tc_mlp(p0, p1, ego, w_gc, b_gc, w_bi, b_bi):
  row_spec = pl.BlockSpec((_BN, _D), lambda i: (i, 0))
  full_w = pl.BlockSpec((_D, _D), lambda i: (0, 0))
  full_b = pl.BlockSpec((1, _D), lambda i: (0, 0))
  return pl.pallas_call(
      _tc_body,
      grid=(_N // _BN,),
      in_specs=[row_spec, row_spec, row_spec, full_w, full_b, full_w, full_b],
      out_specs=row_spec,
      out_shape=jax.ShapeDtypeStruct((_N, _D), jnp.float32),
  )(p0, p1, ego, w_gc, b_gc, w_bi, b_bi)


@jax.jit
def kernel(edge_index, edge_vals, ego_embeddings, w_gc, b_gc, w_bi, b_bi):
  edge4d = edge_index.astype(jnp.int32).reshape(2, _NW * _NSB, _CPS, _CH)
  src = edge_index[0].astype(jnp.int32)
  p0, p1 = _sc_segment_sum(src, edge4d, edge_vals, ego_embeddings)
  return _tc_mlp(p0, p1, ego_embeddings,
                 w_gc, b_gc.reshape(1, _D), w_bi, b_bi.reshape(1, _D))
